# Initial kernel scaffold; baseline (speedup 1.0000x reference)
import jax
import jax.numpy as jnp
from jax.experimental import pallas as pl


def kernel(input_ids, token_table, pos_table):
    raise NotImplementedError("write your pallas kernel here")


# speedup vs baseline: 2.7613x; 2.7613x over previous
import functools

import jax
import jax.numpy as jnp
from jax import lax
from jax.experimental import pallas as pl
from jax.experimental.pallas import tpu as pltpu
from jax.experimental.pallas import tpu_sc as plsc

VOCAB = 1000000
D = 64
B = 4096
S = 200
FLAT = B * S
IDX_COLS = 100
IDX_ROWS = FLAT // IDX_COLS
NC, NS = 2, 16
NW = NC * NS
ROWS_PW = IDX_ROWS // NW
CHUNK_IDX_ROWS = 4
CHUNK = CHUNK_IDX_ROWS * IDX_COLS
NCH = ROWS_PW // CHUNK_IDX_ROWS
LANES = 16

_mesh = plsc.VectorSubcoreMesh(core_axis_name="c", subcore_axis_name="s")


@functools.partial(
    pl.kernel,
    mesh=_mesh,
    out_type=jax.ShapeDtypeStruct((FLAT, D), jnp.float32),
    scratch_types=[
        pltpu.VMEM((ROWS_PW, IDX_COLS), jnp.int32),
        pltpu.VMEM((CHUNK, D), jnp.float32),
        pltpu.VMEM((CHUNK, D), jnp.float32),
        pltpu.VMEM((CHUNK, D), jnp.float32),
        pltpu.SemaphoreType.DMA,
        pltpu.SemaphoreType.DMA,
    ],
    compiler_params=pltpu.CompilerParams(use_tc_tiling_on_sc=False),
)
def _embed_sc(ids_hbm, tab_hbm, pos_hbm, out_hbm,
              idx_v, buf0, buf1, pos_v, sem0, sem1):
    wid = lax.axis_index("s") * NC + lax.axis_index("c")
    row0 = wid * ROWS_PW
    out0 = wid * (ROWS_PW * IDX_COLS)

    pltpu.sync_copy(ids_hbm.at[pl.ds(row0, ROWS_PW)], idx_v)
    pltpu.sync_copy(pos_hbm.at[pl.ds(0, S)], pos_v.at[pl.ds(0, S)])
    pltpu.sync_copy(pos_hbm.at[pl.ds(0, S)], pos_v.at[pl.ds(S, S)])

    def fire(g, buf, sem):
        r0 = g * CHUNK_IDX_ROWS
        for j in range(CHUNK_IDX_ROWS):
            pltpu.async_copy(
                tab_hbm.at[idx_v.at[r0 + j]],
                buf.at[pl.ds(j * IDX_COLS, IDX_COLS)],
                sem,
            )

    def drain(buf, sem):
        pltpu.make_async_copy(tab_hbm.at[pl.ds(0, CHUNK)], buf, sem).wait()

    def add_pos(buf):
        def body(r, carry):
            for c in range(D // LANES):
                sl = pl.ds(c * LANES, LANES)
                buf[r, sl] = buf[r, sl] + pos_v[r, sl]
            return carry
        lax.fori_loop(0, CHUNK, body, 0)

    def writeback(buf, g):
        pltpu.sync_copy(buf, out_hbm.at[pl.ds(out0 + g * CHUNK, CHUNK)])

    fire(0, buf0, sem0)

    def chunk_pair(i, carry):
        g = 2 * i
        fire(g + 1, buf1, sem1)
        drain(buf0, sem0)
        add_pos(buf0)
        writeback(buf0, g)

        @pl.when(g + 2 < NCH)
        def _():
            fire(g + 2, buf0, sem0)

        drain(buf1, sem1)
        add_pos(buf1)
        writeback(buf1, g + 1)
        return carry

    lax.fori_loop(0, NCH // 2, chunk_pair, 0)


def kernel(input_ids, token_table, pos_table):
    ids2d = input_ids.reshape(IDX_ROWS, IDX_COLS).astype(jnp.int32)
    out = _embed_sc(ids2d, token_table, pos_table)
    return out.reshape(B, S, D)


# speedup vs baseline: 2.8154x; 1.0196x over previous
import functools

import jax
import jax.numpy as jnp
from jax import lax
from jax.experimental import pallas as pl
from jax.experimental.pallas import tpu as pltpu
from jax.experimental.pallas import tpu_sc as plsc

VOCAB = 1000000
D = 64
B = 4096
S = 200
SPLITS = ((0, 104), (104, 96))
NC, NS = 2, 16
NW = NC * NS
NB_PW = B // NW
CHUNK_B = 2
NCH = NB_PW // CHUNK_B
LANES = 16

_mesh = plsc.VectorSubcoreMesh(core_axis_name="c", subcore_axis_name="s")


@functools.partial(
    pl.kernel,
    mesh=_mesh,
    out_type=jax.ShapeDtypeStruct((B, S, D), jnp.float32),
    scratch_types=[
        pltpu.VMEM((NB_PW, S), jnp.int32),
        pltpu.VMEM((CHUNK_B, S, D), jnp.float32),
        pltpu.VMEM((CHUNK_B, S, D), jnp.float32),
        pltpu.VMEM((S, D), jnp.float32),
        pltpu.SemaphoreType.DMA,
        pltpu.SemaphoreType.DMA,
    ],
    compiler_params=pltpu.CompilerParams(use_tc_tiling_on_sc=False),
)
def _embed_sc(ids_hbm, tab_hbm, pos_hbm, out_hbm,
              idx_v, buf0, buf1, pos_v, sem0, sem1):
    wid = lax.axis_index("s") * NC + lax.axis_index("c")
    b0w = wid * NB_PW

    pltpu.sync_copy(ids_hbm.at[pl.ds(b0w, NB_PW)], idx_v)
    pltpu.sync_copy(pos_hbm.at[pl.ds(0, S)], pos_v)

    def fire(g, buf, sem):
        bb = g * CHUNK_B
        for p in range(CHUNK_B):
            for off, n in SPLITS:
                pltpu.async_copy(
                    tab_hbm.at[idx_v.at[bb + p, pl.ds(off, n)]],
                    buf.at[p, pl.ds(off, n)],
                    sem,
                )

    def drain(buf, sem):
        for p in range(CHUNK_B):
            for off, n in SPLITS:
                pltpu.make_async_copy(
                    tab_hbm.at[pl.ds(0, n)],
                    buf.at[p, pl.ds(off, n)],
                    sem,
                ).wait()

    def add_pos(buf):
        def body(r, carry):
            for p in range(CHUNK_B):
                for c in range(D // LANES):
                    sl = pl.ds(c * LANES, LANES)
                    buf[p, r, sl] = buf[p, r, sl] + pos_v[r, sl]
            return carry
        lax.fori_loop(0, S, body, 0)

    def writeback(buf, g):
        pltpu.sync_copy(buf, out_hbm.at[pl.ds(b0w + g * CHUNK_B, CHUNK_B)])

    fire(0, buf0, sem0)

    def chunk_pair(i, carry):
        g = 2 * i
        fire(g + 1, buf1, sem1)
        drain(buf0, sem0)
        add_pos(buf0)
        writeback(buf0, g)

        @pl.when(g + 2 < NCH)
        def _():
            fire(g + 2, buf0, sem0)

        drain(buf1, sem1)
        add_pos(buf1)
        writeback(buf1, g + 1)
        return carry

    lax.fori_loop(0, NCH // 2, chunk_pair, 0)


def kernel(input_ids, token_table, pos_table):
    return _embed_sc(input_ids.astype(jnp.int32), token_table, pos_table)
